# baseline (speedup 1.0000x reference)
import jax
import jax.numpy as jnp
from jax import lax
from jax.experimental import pallas as pl
from jax.experimental.pallas import tpu as pltpu
from jax.experimental.pallas import tpu_sc as plsc

N = 10000
E = 160000
IN_DIM = 256
HID_DIM = 512
OUT_DIM = 256

NC = 2
NS = 16
B = 128
NB = 80
IH = NB // 2
E_PAD = NS * NB * B
RPT = 632
N_PAD = NS * RPT
C = 128
NSLOT = 2

R = 1024
G = 10

NCH1 = IN_DIM // C
NCH2 = HID_DIM // C


def _make_sc_agg(n_chunks, with_cnt):
  mesh = plsc.VectorSubcoreMesh(core_axis_name="c", subcore_axis_name="s")
  out_type = [jax.ShapeDtypeStruct((N_PAD, C), jnp.float32)
              for _ in range(n_chunks)]
  if with_cnt:
    out_type.append(jax.ShapeDtypeStruct((N_PAD,), jnp.float32))
  scratch = [
      pltpu.VMEM((IH, B), jnp.int32),
      pltpu.VMEM((IH, B), jnp.int32),
      pltpu.VMEM((NSLOT, B, C), jnp.float32),
      pltpu.VMEM_SHARED((N_PAD, C), jnp.float32),
      pltpu.SemaphoreType.DMA,
      pltpu.SemaphoreType.DMA,
  ]
  if with_cnt:
    scratch += [
        pltpu.VMEM((B,), jnp.float32),
        pltpu.VMEM((RPT,), jnp.float32),
        pltpu.VMEM_SHARED((N_PAD,), jnp.float32),
        pltpu.SemaphoreType.DMA,
    ]

  def body(*args):
    a = list(args)
    src_hbm, dst_hbm, z2d_hbm = a[:3]
    a = a[3:]
    if with_cnt:
      ones_hbm, z1d_hbm = a[:2]
      a = a[2:]
    tables = a[:n_chunks]
    a = a[n_chunks:]
    outs = a[:n_chunks]
    a = a[n_chunks:]
    if with_cnt:
      cnt_out = a[0]
      a = a[1:]
    src_v, dst_v, rowsr_v, acc_sh, sem_g, sem_s = a[:6]
    if with_cnt:
      ones_v, z1d_v, cnt_sh, sem_c = a[6:10]

    c = lax.axis_index("c")
    s = lax.axis_index("s")
    row0 = s * RPT

    if with_cnt:
      pltpu.sync_copy(ones_hbm, ones_v)
      pltpu.sync_copy(z1d_hbm, z1d_v)

    for k in range(n_chunks):
      @pl.when(c == (k % NC))
      def _chunk(k=k):
        pltpu.sync_copy(z2d_hbm, rowsr_v.at[0])
        for j in range(RPT // 128):
          pltpu.sync_copy(rowsr_v.at[0], acc_sh.at[pl.ds(row0 + j * 128, 128)])
        rem = RPT % 128
        if rem:
          pltpu.sync_copy(rowsr_v.at[0].at[pl.ds(0, rem)],
                          acc_sh.at[pl.ds(row0 + (RPT // 128) * 128, rem)])
        if with_cnt and k == 0:
          pltpu.sync_copy(z1d_v, cnt_sh.at[pl.ds(row0, RPT)])
        plsc.subcore_barrier()

        for half in range(NB // IH):
          pltpu.sync_copy(src_hbm.at[s].at[pl.ds(half * IH, IH)], src_v)
          pltpu.sync_copy(dst_hbm.at[s].at[pl.ds(half * IH, IH)], dst_v)
          pltpu.async_copy(tables[k].at[src_v.at[0]], rowsr_v.at[0], sem_g)

          def step(b, carry, k=k, half=half):
            slot = lax.rem(b, NSLOT)
            nslot = lax.rem(b + 1, NSLOT)

            @pl.when(b >= 1)
            def _drain():
              pltpu.make_async_copy(rowsr_v.at[nslot],
                                    acc_sh.at[dst_v.at[b - 1]], sem_s).wait()

            @pl.when(b + 1 < IH)
            def _prefetch():
              pltpu.async_copy(tables[k].at[src_v.at[b + 1]],
                               rowsr_v.at[nslot], sem_g)

            pltpu.make_async_copy(tables[k].at[src_v.at[b]],
                                  rowsr_v.at[slot], sem_g).wait()
            pltpu.async_copy(rowsr_v.at[slot],
                             acc_sh.at[dst_v.at[b]], sem_s, add=True)
            if with_cnt and k == 0:
              pltpu.async_copy(ones_v, cnt_sh.at[dst_v.at[b]], sem_c, add=True)
            return carry

          lax.fori_loop(0, IH, step, 0)
          pltpu.make_async_copy(rowsr_v.at[(IH - 1) % NSLOT],
                                acc_sh.at[dst_v.at[IH - 1]], sem_s).wait()
          if with_cnt and k == 0:
            def drain_cnt(b, carry):
              pltpu.make_async_copy(ones_v, cnt_sh.at[dst_v.at[b]],
                                    sem_c).wait()
              return carry
            lax.fori_loop(0, IH, drain_cnt, 0)
        plsc.subcore_barrier()

        for j in range(RPT // 128):
          pltpu.sync_copy(acc_sh.at[pl.ds(row0 + j * 128, 128)],
                          rowsr_v.at[0])
          pltpu.sync_copy(rowsr_v.at[0], outs[k].at[pl.ds(row0 + j * 128, 128)])
        if rem:
          pltpu.sync_copy(acc_sh.at[pl.ds(row0 + (RPT // 128) * 128, rem)],
                          rowsr_v.at[0].at[pl.ds(0, rem)])
          pltpu.sync_copy(rowsr_v.at[0].at[pl.ds(0, rem)],
                          outs[k].at[pl.ds(row0 + (RPT // 128) * 128, rem)])
        if with_cnt and k == 0:
          pltpu.sync_copy(cnt_sh.at[pl.ds(row0, RPT)], z1d_v)
          pltpu.sync_copy(z1d_v, cnt_out.at[pl.ds(row0, RPT)])

    return None

  return pl.kernel(
      body, out_type=out_type, mesh=mesh, scratch_types=scratch,
      compiler_params=pltpu.CompilerParams(use_tc_tiling_on_sc=False))


_sc_agg_l1 = _make_sc_agg(NCH1, with_cnt=True)
_sc_agg_l2 = _make_sc_agg(NCH2, with_cnt=False)


def _tc_layer1(aggs, cnt, x, wl, bl, wr):
  def bodyfn(*refs):
    a_refs = refs[:NCH1]
    cnt_ref, x_ref, wl_ref, bl_ref, wr_ref = refs[NCH1:NCH1 + 5]
    h_refs = refs[NCH1 + 5:]
    inv = 1.0 / jnp.maximum(cnt_ref[...], 1.0)
    agg = jnp.concatenate([r[...] for r in a_refs], axis=1) * inv
    h = jnp.dot(agg, wl_ref[...], preferred_element_type=jnp.float32)
    h = h + jnp.dot(x_ref[...], wr_ref[...], preferred_element_type=jnp.float32)
    h = jnp.maximum(h + bl_ref[...], 0.0)
    for k, hr in enumerate(h_refs):
      hr[...] = h[:, k * C:(k + 1) * C]

  return pl.pallas_call(
      bodyfn,
      grid=(G,),
      in_specs=[pl.BlockSpec((R, C), lambda i: (i, 0))] * NCH1 + [
          pl.BlockSpec((R, 1), lambda i: (i, 0)),
          pl.BlockSpec((R, IN_DIM), lambda i: (i, 0)),
          pl.BlockSpec((IN_DIM, HID_DIM), lambda i: (0, 0)),
          pl.BlockSpec((1, HID_DIM), lambda i: (0, 0)),
          pl.BlockSpec((IN_DIM, HID_DIM), lambda i: (0, 0)),
      ],
      out_specs=[pl.BlockSpec((R, C), lambda i: (i, 0))] * NCH2,
      out_shape=[jax.ShapeDtypeStruct((N_PAD, C), jnp.float32)] * NCH2,
  )(*aggs, cnt, x, wl, bl, wr)


def _tc_layer2(aggs, cnt, hs, wl, bl, wr):
  def bodyfn(*refs):
    a_refs = refs[:NCH2]
    cnt_ref = refs[NCH2]
    h_refs = refs[NCH2 + 1:2 * NCH2 + 1]
    wl_ref, bl_ref, wr_ref = refs[2 * NCH2 + 1:2 * NCH2 + 4]
    out_ref = refs[-1]
    inv = 1.0 / jnp.maximum(cnt_ref[...], 1.0)
    agg = jnp.concatenate([r[...] for r in a_refs], axis=1) * inv
    h = jnp.concatenate([r[...] for r in h_refs], axis=1)
    o = jnp.dot(agg, wl_ref[...], preferred_element_type=jnp.float32)
    o = o + jnp.dot(h, wr_ref[...], preferred_element_type=jnp.float32)
    out_ref[...] = o + bl_ref[...]

  return pl.pallas_call(
      bodyfn,
      grid=(G,),
      in_specs=[pl.BlockSpec((R, C), lambda i: (i, 0))] * NCH2 + [
          pl.BlockSpec((R, 1), lambda i: (i, 0)),
      ] + [pl.BlockSpec((R, C), lambda i: (i, 0))] * NCH2 + [
          pl.BlockSpec((HID_DIM, OUT_DIM), lambda i: (0, 0)),
          pl.BlockSpec((1, OUT_DIM), lambda i: (0, 0)),
          pl.BlockSpec((HID_DIM, OUT_DIM), lambda i: (0, 0)),
      ],
      out_specs=pl.BlockSpec((R, OUT_DIM), lambda i: (i, 0)),
      out_shape=jax.ShapeDtypeStruct((N, OUT_DIM), jnp.float32),
  )(*aggs, cnt, *hs, wl, bl, wr)


def kernel(x, edge_index, Wl1, bl1, Wr1, Wl2, bl2, Wr2):
  ei = edge_index.astype(jnp.int32)
  src = jnp.concatenate(
      [ei[0], jnp.zeros((E_PAD - E,), jnp.int32)]).reshape(NS, NB, B)
  dst = jnp.concatenate(
      [ei[1], jnp.full((E_PAD - E,), N, jnp.int32)]).reshape(NS, NB, B)
  z2d = jnp.zeros((128, C), jnp.float32)
  ones1 = jnp.ones((B,), jnp.float32)
  z1d = jnp.zeros((RPT,), jnp.float32)

  xc = [x[:, k * C:(k + 1) * C] for k in range(NCH1)]
  *a, cnt = _sc_agg_l1(src, dst, z2d, ones1, z1d, *xc)
  cnt2 = cnt.reshape(N_PAD, 1)

  h = _tc_layer1(a, cnt2, x, Wl1.T, bl1.reshape(1, -1), Wr1.T)

  g = _sc_agg_l2(src, dst, z2d, *h)

  return _tc_layer2(g, cnt2, h, Wl2.T, bl2.reshape(1, -1), Wr2.T)
